# baseline (speedup 1.0000x reference)
import functools

import jax
import jax.numpy as jnp
from jax import lax
from jax.experimental import pallas as pl
from jax.experimental.pallas import tpu as pltpu
from jax.experimental.pallas import tpu_sc as plsc

D = 128
_info = plsc.get_sparse_core_info()
NC, NS = _info.num_cores, _info.num_subcores
NW = NC * NS
NBUF = 5


@jax.jit
def _sc_gather(table, idx_t):
    n_pos, n_batch = idx_t.shape
    bpw = n_batch // NW
    n_chunks = n_pos
    n_groups = n_chunks // NBUF

    @functools.partial(
        pl.kernel,
        mesh=plsc.VectorSubcoreMesh(core_axis_name="c", subcore_axis_name="s"),
        out_type=jax.ShapeDtypeStruct((n_pos, n_batch, D), jnp.float32),
        scratch_types=[
            pltpu.VMEM((n_pos, bpw), jnp.int32),
            pltpu.VMEM((NBUF, bpw, D), jnp.float32),
            pltpu.SemaphoreType.DMA((NBUF,)),
            pltpu.SemaphoreType.DMA((NBUF,)),
        ],
    )
    def k(table_hbm, idx_hbm, out_hbm, idx_v, rows, gsem, ssem):
        wid = lax.axis_index("s") * NC + lax.axis_index("c")
        b0 = wid * bpw
        pltpu.sync_copy(idx_hbm.at[:, pl.ds(b0, bpw)], idx_v)

        def start_g(j, b):
            pltpu.async_copy(table_hbm.at[idx_v.at[j]], rows.at[b], gsem.at[b])

        def wait_g(j, b):
            pltpu.make_async_copy(
                table_hbm.at[idx_v.at[j]], rows.at[b], gsem.at[b]
            ).wait()

        def start_s(j, b):
            pltpu.async_copy(rows.at[b], out_hbm.at[j, pl.ds(b0, bpw)], ssem.at[b])

        def wait_s(j, b):
            pltpu.make_async_copy(
                rows.at[b], out_hbm.at[j, pl.ds(b0, bpw)], ssem.at[b]
            ).wait()

        def step(j, u, wait_prev, do_next):
            wait_g(j, u)
            start_s(j, u)
            if do_next:
                bp = (u - 1) % NBUF
                if wait_prev:
                    wait_s(j - 1, bp)
                start_g(j + NBUF - 1, bp)
            elif wait_prev:
                wait_s(j - 1, (u - 1) % NBUF)

        for u in range(NBUF - 1):
            start_g(u, u)
        for u in range(NBUF):
            step(u, u, wait_prev=(u > 0), do_next=True)
        def body(g, c):
            for u in range(NBUF):
                step(NBUF * g + u, u, wait_prev=True, do_next=True)
            return c
        lax.fori_loop(1, n_groups - 1, body, 0)
        jt = NBUF * (n_groups - 1)
        for u in range(NBUF):
            step(jt + u, u, wait_prev=True, do_next=(jt + u + NBUF - 1 < n_chunks))
        for j in range(NBUF * n_groups, n_chunks):
            step(j, j % NBUF, wait_prev=True, do_next=False)
        wait_s(n_chunks - 1, (n_chunks - 1) % NBUF)

    return k(table, idx_t)


def kernel(action_ids, embedding):
    idx_t = jnp.swapaxes(action_ids.astype(jnp.int32), 0, 1)
    out = _sc_gather(embedding, idx_t)
    return jnp.transpose(out, (1, 0, 2))


# speedup vs baseline: 1.0014x; 1.0014x over previous
import functools

import jax
import jax.numpy as jnp
from jax import lax
from jax.experimental import pallas as pl
from jax.experimental.pallas import tpu as pltpu
from jax.experimental.pallas import tpu_sc as plsc

D = 128
_info = plsc.get_sparse_core_info()
NC, NS = _info.num_cores, _info.num_subcores
NW = NC * NS
NBUF = 5


@jax.jit
def _sc_gather(table, idx_t):
    n_pos, n_batch = idx_t.shape
    bpw = n_batch // NW
    n_chunks = n_pos
    n_groups = n_chunks // NBUF

    @functools.partial(
        pl.kernel,
        mesh=plsc.VectorSubcoreMesh(core_axis_name="c", subcore_axis_name="s"),
        out_type=jax.ShapeDtypeStruct((n_pos, n_batch, D), jnp.float32),
        compiler_params=pltpu.CompilerParams(
            disable_bounds_checks=True, disable_semaphore_checks=True
        ),
        scratch_types=[
            pltpu.VMEM((n_pos, bpw), jnp.int32),
            pltpu.VMEM((NBUF, bpw, D), jnp.float32),
            pltpu.SemaphoreType.DMA((NBUF,)),
            pltpu.SemaphoreType.DMA((NBUF,)),
        ],
    )
    def k(table_hbm, idx_hbm, out_hbm, idx_v, rows, gsem, ssem):
        wid = lax.axis_index("s") * NC + lax.axis_index("c")
        b0 = wid * bpw
        pltpu.sync_copy(idx_hbm.at[:, pl.ds(b0, bpw)], idx_v)

        def start_g(j, b):
            pltpu.async_copy(table_hbm.at[idx_v.at[j]], rows.at[b], gsem.at[b])

        def wait_g(j, b):
            pltpu.make_async_copy(
                table_hbm.at[idx_v.at[j]], rows.at[b], gsem.at[b]
            ).wait()

        def start_s(j, b):
            pltpu.async_copy(rows.at[b], out_hbm.at[j, pl.ds(b0, bpw)], ssem.at[b])

        def wait_s(j, b):
            pltpu.make_async_copy(
                rows.at[b], out_hbm.at[j, pl.ds(b0, bpw)], ssem.at[b]
            ).wait()

        def step(j, u, wait_prev, do_next):
            wait_g(j, u)
            start_s(j, u)
            if do_next:
                bp = (u - 1) % NBUF
                if wait_prev:
                    wait_s(j - 1, bp)
                start_g(j + NBUF - 1, bp)
            elif wait_prev:
                wait_s(j - 1, (u - 1) % NBUF)

        for u in range(NBUF - 1):
            start_g(u, u)
        for u in range(NBUF):
            step(u, u, wait_prev=(u > 0), do_next=True)
        def body(g, c):
            for u in range(NBUF):
                step(NBUF * g + u, u, wait_prev=True, do_next=True)
            return c
        lax.fori_loop(1, n_groups - 1, body, 0)
        jt = NBUF * (n_groups - 1)
        for u in range(NBUF):
            step(jt + u, u, wait_prev=True, do_next=(jt + u + NBUF - 1 < n_chunks))
        for j in range(NBUF * n_groups, n_chunks):
            step(j, j % NBUF, wait_prev=True, do_next=False)
        wait_s(n_chunks - 1, (n_chunks - 1) % NBUF)

    return k(table, idx_t)


def kernel(action_ids, embedding):
    idx_t = jnp.swapaxes(action_ids.astype(jnp.int32), 0, 1)
    out = _sc_gather(embedding, idx_t)
    return jnp.transpose(out, (1, 0, 2))
